# baseline (speedup 1.0000x reference)
import functools

import jax
import jax.numpy as jnp
from jax import lax
from jax.experimental import pallas as pl
from jax.experimental.pallas import tpu as pltpu
from jax.experimental.pallas import tpu_sc as plsc

_BATCH = 4096
_HIST = 50
_D = 64
_NL = 1000000
_BL = 256
_S = 500224
_NBLK = _S // _BL
_SROWS = 2 * _S


@functools.lru_cache(maxsize=None)
def _build_transpose():
    def body(lo_ref, hi_ref, out_ref):
        eye = (lax.broadcasted_iota(jnp.int32, (_D, _D), 0) ==
               lax.broadcasted_iota(jnp.int32, (_D, _D), 1)
               ).astype(jnp.float32)
        dn = (((0,), (0,)), ((), ()))
        lo = lax.dot_general(lo_ref[...], eye, dn,
                             preferred_element_type=jnp.float32)
        hi = lax.dot_general(hi_ref[...], eye, dn,
                             preferred_element_type=jnp.float32)
        out_ref[...] = jnp.concatenate([lo, hi], axis=1)

    return pl.pallas_call(
        body,
        grid=(_NBLK,),
        in_specs=[
            pl.BlockSpec((_D, _BL), lambda j: (0, j)),
            pl.BlockSpec((_D, _BL),
                         lambda j: (0, jnp.minimum(j + _NBLK,
                                                   (_NL - 1) // _BL))),
        ],
        out_specs=pl.BlockSpec((_BL, 128), lambda j: (j, 0)),
        out_shape=jax.ShapeDtypeStruct((_S, 128), jnp.float32),
        compiler_params=pltpu.CompilerParams(
            dimension_semantics=("arbitrary",)),
    )


@functools.lru_cache(maxsize=None)
def _build_gather():
    info = plsc.get_sparse_core_info()
    nc, ns = info.num_cores, info.num_subcores
    nw = nc * ns
    ub = _BATCH // nw
    ib = _BATCH * _HIST // nw
    chunk = 128
    nchunk = ib // chunk
    nbuf = 5
    ngrp = nchunk // nbuf

    mesh = plsc.VectorSubcoreMesh(core_axis_name="c", subcore_axis_name="s")

    @functools.partial(
        pl.kernel,
        out_type=(
            jax.ShapeDtypeStruct((_BATCH, _D), jnp.float32),
            jax.ShapeDtypeStruct((_BATCH * _HIST, _D), jnp.float32),
        ),
        mesh=mesh,
        compiler_params=pltpu.CompilerParams(use_tc_tiling_on_sc=False),
        scratch_types=[
            pltpu.VMEM((ub,), jnp.int32),
            pltpu.VMEM((ub, _D), jnp.float32),
            pltpu.VMEM((ib,), jnp.int32),
            [pltpu.VMEM((chunk, _D), jnp.float32) for _ in range(nbuf)],
            pltpu.SemaphoreType.DMA,
            [pltpu.SemaphoreType.DMA for _ in range(nbuf)],
            [pltpu.SemaphoreType.DMA for _ in range(nbuf)],
        ],
    )
    def emb(uid, iid, utab, itab, uout, iout,
            uidx, urows, iidx, bufs, usem, gsems, wsems):
        wid = lax.axis_index("s") * nc + lax.axis_index("c")
        ubase = wid * ub
        ibase = wid * ib

        pltpu.sync_copy(uid.at[pl.ds(ubase, ub)], uidx)

        @pl.loop(0, ub // 16)
        def _tu(t):
            iv = uidx[pl.ds(t * 16, 16)]
            uidx[pl.ds(t * 16, 16)] = jnp.where(
                iv < _S, 2 * iv, 2 * (iv - _S) + 1)

        pltpu.async_copy(utab.at[uidx], urows, usem)
        pltpu.sync_copy(iid.at[pl.ds(ibase, ib)], iidx)

        @pl.loop(0, ib // 16)
        def _ti(t):
            iv = iidx[pl.ds(t * 16, 16)]
            iidx[pl.ds(t * 16, 16)] = jnp.where(
                iv < _S, 2 * iv, 2 * (iv - _S) + 1)

        pltpu.make_async_copy(utab.at[uidx], urows, usem).wait()
        pltpu.async_copy(urows, uout.at[pl.ds(ubase, ub)], usem)

        def gather(c, b):
            pltpu.async_copy(
                itab.at[iidx.at[pl.ds(c * chunk, chunk)]], bufs[b], gsems[b])

        def wait_gather(c, b):
            pltpu.make_async_copy(
                itab.at[iidx.at[pl.ds(c * chunk, chunk)]], bufs[b],
                gsems[b]).wait()

        def put(c, b):
            pltpu.async_copy(
                bufs[b], iout.at[pl.ds(ibase + c * chunk, chunk)], wsems[b])

        def wait_put(c, b):
            pltpu.make_async_copy(
                bufs[b], iout.at[pl.ds(ibase + c * chunk, chunk)],
                wsems[b]).wait()

        for b in range(nbuf):
            gather(b, b)

        @pl.loop(0, ngrp - 1)
        def _grp(g):
            c0 = g * nbuf
            for b in range(nbuf):
                wait_gather(c0 + b, b)
                put(c0 + b, b)
                wait_put(c0 + b, b)
                gather(c0 + nbuf + b, b)

        c0 = (ngrp - 1) * nbuf
        for b in range(nbuf):
            wait_gather(c0 + b, b)
            put(c0 + b, b)
            wait_put(c0 + b, b)

        pltpu.make_async_copy(urows, uout.at[pl.ds(ubase, ub)], usem).wait()

    return emb


def kernel(user_id, items_ids, user_table, item_table):
    tposer = _build_transpose()
    emb = _build_gather()
    uid = user_id.astype(jnp.int32)
    iid = items_ids.reshape(-1).astype(jnp.int32)
    iscr = tposer(item_table.T, item_table.T)
    uscr = tposer(user_table.T, user_table.T)
    user_eb, item_flat = emb(uid, iid,
                             uscr.reshape(_SROWS, _D), iscr.reshape(_SROWS, _D))
    return user_eb, item_flat.reshape(_BATCH, _HIST, _D)


# speedup vs baseline: 2.2994x; 2.2994x over previous
import functools

import jax
import jax.numpy as jnp
from jax import lax
from jax.experimental import pallas as pl
from jax.experimental.pallas import tpu as pltpu
from jax.experimental.pallas import tpu_sc as plsc

_BATCH = 4096
_HIST = 50
_D = 64
_NL = 1000000
_BL = 4096
_S = 503808
_NBLK = _S // _BL
_SROWS = 2 * _S


@functools.lru_cache(maxsize=None)
def _build_transpose():
    def body(lo_ref, hi_ref, out_ref):
        eye = (lax.broadcasted_iota(jnp.int32, (_D, _D), 0) ==
               lax.broadcasted_iota(jnp.int32, (_D, _D), 1)
               ).astype(jnp.float32)
        dn = (((0,), (0,)), ((), ()))
        lo = lax.dot_general(lo_ref[...], eye, dn,
                             precision=lax.Precision.HIGHEST,
                             preferred_element_type=jnp.float32)
        hi = lax.dot_general(hi_ref[...], eye, dn,
                             precision=lax.Precision.HIGHEST,
                             preferred_element_type=jnp.float32)
        out_ref[...] = jnp.concatenate([lo, hi], axis=1)

    return pl.pallas_call(
        body,
        grid=(_NBLK,),
        in_specs=[
            pl.BlockSpec((_D, _BL), lambda j: (0, j)),
            pl.BlockSpec((_D, _BL),
                         lambda j: (0, jnp.minimum(j + _NBLK,
                                                   (_NL - 1) // _BL))),
        ],
        out_specs=pl.BlockSpec((_BL, 128), lambda j: (j, 0)),
        out_shape=jax.ShapeDtypeStruct((_S, 128), jnp.float32),
        compiler_params=pltpu.CompilerParams(
            dimension_semantics=("arbitrary",)),
    )


@functools.lru_cache(maxsize=None)
def _build_gather():
    info = plsc.get_sparse_core_info()
    nc, ns = info.num_cores, info.num_subcores
    nw = nc * ns
    ub = _BATCH // nw
    ib = _BATCH * _HIST // nw
    chunk = 128
    nchunk = ib // chunk
    nbuf = 5
    ngrp = nchunk // nbuf

    mesh = plsc.VectorSubcoreMesh(core_axis_name="c", subcore_axis_name="s")

    @functools.partial(
        pl.kernel,
        out_type=(
            jax.ShapeDtypeStruct((_BATCH, _D), jnp.float32),
            jax.ShapeDtypeStruct((_BATCH * _HIST, _D), jnp.float32),
        ),
        mesh=mesh,
        compiler_params=pltpu.CompilerParams(use_tc_tiling_on_sc=False),
        scratch_types=[
            pltpu.VMEM((ub,), jnp.int32),
            pltpu.VMEM((ub, _D), jnp.float32),
            pltpu.VMEM((ib,), jnp.int32),
            [pltpu.VMEM((chunk, _D), jnp.float32) for _ in range(nbuf)],
            pltpu.SemaphoreType.DMA,
            [pltpu.SemaphoreType.DMA for _ in range(nbuf)],
            [pltpu.SemaphoreType.DMA for _ in range(nbuf)],
        ],
    )
    def emb(uid, iid, utab, itab, uout, iout,
            uidx, urows, iidx, bufs, usem, gsems, wsems):
        wid = lax.axis_index("s") * nc + lax.axis_index("c")
        ubase = wid * ub
        ibase = wid * ib

        pltpu.sync_copy(uid.at[pl.ds(ubase, ub)], uidx)

        @pl.loop(0, ub // 16)
        def _tu(t):
            iv = uidx[pl.ds(t * 16, 16)]
            uidx[pl.ds(t * 16, 16)] = jnp.where(
                iv < _S, 2 * iv, 2 * (iv - _S) + 1)

        pltpu.async_copy(utab.at[uidx], urows, usem)
        pltpu.sync_copy(iid.at[pl.ds(ibase, ib)], iidx)

        @pl.loop(0, ib // 16)
        def _ti(t):
            iv = iidx[pl.ds(t * 16, 16)]
            iidx[pl.ds(t * 16, 16)] = jnp.where(
                iv < _S, 2 * iv, 2 * (iv - _S) + 1)

        pltpu.make_async_copy(utab.at[uidx], urows, usem).wait()
        pltpu.async_copy(urows, uout.at[pl.ds(ubase, ub)], usem)

        def gather(c, b):
            pltpu.async_copy(
                itab.at[iidx.at[pl.ds(c * chunk, chunk)]], bufs[b], gsems[b])

        def wait_gather(c, b):
            pltpu.make_async_copy(
                itab.at[iidx.at[pl.ds(c * chunk, chunk)]], bufs[b],
                gsems[b]).wait()

        def put(c, b):
            pltpu.async_copy(
                bufs[b], iout.at[pl.ds(ibase + c * chunk, chunk)], wsems[b])

        def wait_put(c, b):
            pltpu.make_async_copy(
                bufs[b], iout.at[pl.ds(ibase + c * chunk, chunk)],
                wsems[b]).wait()

        for b in range(nbuf):
            gather(b, b)

        @pl.loop(0, ngrp - 1)
        def _grp(g):
            c0 = g * nbuf
            for b in range(nbuf):
                wait_gather(c0 + b, b)
                put(c0 + b, b)
                wait_put(c0 + b, b)
                gather(c0 + nbuf + b, b)

        c0 = (ngrp - 1) * nbuf
        for b in range(nbuf):
            wait_gather(c0 + b, b)
            put(c0 + b, b)
            wait_put(c0 + b, b)

        pltpu.make_async_copy(urows, uout.at[pl.ds(ubase, ub)], usem).wait()

    return emb


def kernel(user_id, items_ids, user_table, item_table):
    tposer = _build_transpose()
    emb = _build_gather()
    uid = user_id.astype(jnp.int32)
    iid = items_ids.reshape(-1).astype(jnp.int32)
    iscr = tposer(item_table.T, item_table.T)
    uscr = tposer(user_table.T, user_table.T)
    user_eb, item_flat = emb(uid, iid,
                             uscr.reshape(_SROWS, _D), iscr.reshape(_SROWS, _D))
    return user_eb, item_flat.reshape(_BATCH, _HIST, _D)


# speedup vs baseline: 2.3504x; 1.0222x over previous
import functools

import jax
import jax.numpy as jnp
from jax import lax
from jax.experimental import pallas as pl
from jax.experimental.pallas import tpu as pltpu
from jax.experimental.pallas import tpu_sc as plsc

_BATCH = 4096
_HIST = 50
_D = 64
_NL = 1000000
_BL = 8192
_S = 507904
_NBLK = _S // _BL
_SROWS = 2 * _S


@functools.lru_cache(maxsize=None)
def _build_transpose():
    def body(lo_ref, hi_ref, out_ref):
        eye = (lax.broadcasted_iota(jnp.int32, (_D, _D), 0) ==
               lax.broadcasted_iota(jnp.int32, (_D, _D), 1)
               ).astype(jnp.float32)
        dn = (((0,), (0,)), ((), ()))
        lo = lax.dot_general(lo_ref[...], eye, dn,
                             precision=lax.Precision.HIGHEST,
                             preferred_element_type=jnp.float32)
        hi = lax.dot_general(hi_ref[...], eye, dn,
                             precision=lax.Precision.HIGHEST,
                             preferred_element_type=jnp.float32)
        out_ref[...] = jnp.concatenate([lo, hi], axis=1)

    return pl.pallas_call(
        body,
        grid=(_NBLK,),
        in_specs=[
            pl.BlockSpec((_D, _BL), lambda j: (0, j)),
            pl.BlockSpec((_D, _BL),
                         lambda j: (0, jnp.minimum(j + _NBLK,
                                                   (_NL - 1) // _BL))),
        ],
        out_specs=pl.BlockSpec((_BL, 128), lambda j: (j, 0)),
        out_shape=jax.ShapeDtypeStruct((_S, 128), jnp.float32),
        compiler_params=pltpu.CompilerParams(
            dimension_semantics=("arbitrary",)),
    )


@functools.lru_cache(maxsize=None)
def _build_gather():
    info = plsc.get_sparse_core_info()
    nc, ns = info.num_cores, info.num_subcores
    nw = nc * ns
    ub = _BATCH // nw
    ib = _BATCH * _HIST // nw
    chunk = 128
    nchunk = ib // chunk
    nbuf = 5
    ngrp = nchunk // nbuf

    mesh = plsc.VectorSubcoreMesh(core_axis_name="c", subcore_axis_name="s")

    @functools.partial(
        pl.kernel,
        out_type=(
            jax.ShapeDtypeStruct((_BATCH, _D), jnp.float32),
            jax.ShapeDtypeStruct((_BATCH * _HIST, _D), jnp.float32),
        ),
        mesh=mesh,
        compiler_params=pltpu.CompilerParams(use_tc_tiling_on_sc=False),
        scratch_types=[
            pltpu.VMEM((ub,), jnp.int32),
            pltpu.VMEM((ub, _D), jnp.float32),
            pltpu.VMEM((ib,), jnp.int32),
            [pltpu.VMEM((chunk, _D), jnp.float32) for _ in range(nbuf)],
            pltpu.SemaphoreType.DMA,
            [pltpu.SemaphoreType.DMA for _ in range(nbuf)],
            [pltpu.SemaphoreType.DMA for _ in range(nbuf)],
        ],
    )
    def emb(uid, iid, utab, itab, uout, iout,
            uidx, urows, iidx, bufs, usem, gsems, wsems):
        wid = lax.axis_index("s") * nc + lax.axis_index("c")
        ubase = wid * ub
        ibase = wid * ib

        pltpu.sync_copy(uid.at[pl.ds(ubase, ub)], uidx)

        @pl.loop(0, ub // 16)
        def _tu(t):
            iv = uidx[pl.ds(t * 16, 16)]
            uidx[pl.ds(t * 16, 16)] = jnp.where(
                iv < _S, 2 * iv, 2 * (iv - _S) + 1)

        pltpu.async_copy(utab.at[uidx], urows, usem)
        pltpu.sync_copy(iid.at[pl.ds(ibase, ib)], iidx)

        @pl.loop(0, ib // 16)
        def _ti(t):
            iv = iidx[pl.ds(t * 16, 16)]
            iidx[pl.ds(t * 16, 16)] = jnp.where(
                iv < _S, 2 * iv, 2 * (iv - _S) + 1)

        pltpu.make_async_copy(utab.at[uidx], urows, usem).wait()
        pltpu.async_copy(urows, uout.at[pl.ds(ubase, ub)], usem)

        def gather(c, b):
            pltpu.async_copy(
                itab.at[iidx.at[pl.ds(c * chunk, chunk)]], bufs[b], gsems[b])

        def wait_gather(c, b):
            pltpu.make_async_copy(
                itab.at[iidx.at[pl.ds(c * chunk, chunk)]], bufs[b],
                gsems[b]).wait()

        def put(c, b):
            pltpu.async_copy(
                bufs[b], iout.at[pl.ds(ibase + c * chunk, chunk)], wsems[b])

        def wait_put(c, b):
            pltpu.make_async_copy(
                bufs[b], iout.at[pl.ds(ibase + c * chunk, chunk)],
                wsems[b]).wait()

        for b in range(nbuf):
            gather(b, b)

        @pl.loop(0, ngrp - 1)
        def _grp(g):
            c0 = g * nbuf
            for b in range(nbuf):
                wait_gather(c0 + b, b)
                put(c0 + b, b)
                wait_put(c0 + b, b)
                gather(c0 + nbuf + b, b)

        c0 = (ngrp - 1) * nbuf
        for b in range(nbuf):
            wait_gather(c0 + b, b)
            put(c0 + b, b)
            wait_put(c0 + b, b)

        pltpu.make_async_copy(urows, uout.at[pl.ds(ubase, ub)], usem).wait()

    return emb


def kernel(user_id, items_ids, user_table, item_table):
    tposer = _build_transpose()
    emb = _build_gather()
    uid = user_id.astype(jnp.int32)
    iid = items_ids.reshape(-1).astype(jnp.int32)
    iscr = tposer(item_table.T, item_table.T)
    uscr = tposer(user_table.T, user_table.T)
    user_eb, item_flat = emb(uid, iid,
                             uscr.reshape(_SROWS, _D), iscr.reshape(_SROWS, _D))
    return user_eb, item_flat.reshape(_BATCH, _HIST, _D)
